# baseline (speedup 1.0000x reference)
import jax
import jax.numpy as jnp
from jax import lax
from jax.experimental import pallas as pl
from jax.experimental.pallas import tpu as pltpu
from jax.experimental.pallas import tpu_sc as plsc

N = 10000
D = 128
E = 320000
MAX_ITER = 10
STEP = 0.1 / 1.5
YC = 16
NC = 2
NS = 16
NW = NC * NS
B = 64
BPW = 160
ICH = 16
EPAD = NW * BPW * B
EROWS = EPAD // B
ZCH = 40
NROWCHUNKS = N // ZCH
_NCHUNK_CEIL = -(-NROWCHUNKS // NS)


def _sc_step_kernel():
    mesh = plsc.VectorSubcoreMesh(core_axis_name="c", subcore_axis_name="s",
                                  num_cores=NC, num_subcores=NS)

    def body(state_hbm, y_hbm, src_hbm, dst_hbm, w_hbm, out_hbm,
             acc, xs, xd, ys, yd, en, sidx, didx, wbuf,
             gsem0, gsem1, ssem0, ssem1):
        cid = lax.axis_index("c")
        sid = lax.axis_index("s")
        wid = sid * NC + cid
        gsems = [gsem0, gsem1]
        ssems = [ssem0, ssem1]

        zero16 = jnp.zeros((16,), jnp.float32)

        def zbody(r, carry):
            for j in range(D // 16):
                xs[0, r, pl.ds(16 * j, 16)] = zero16
            return carry

        lax.fori_loop(0, B, zbody, 0)

        for k in range(_NCHUNK_CEIL):
            chunk = sid + k * NS

            @pl.when(chunk < NROWCHUNKS)
            def _():
                pltpu.sync_copy(xs.at[0, pl.ds(0, ZCH)],
                                acc.at[pl.ds(chunk * ZCH, ZCH)])

        pltpu.sync_copy(w_hbm, wbuf)
        pltpu.sync_copy(src_hbm.at[pl.ds(wid * BPW, ICH)], sidx.at[0])
        pltpu.sync_copy(dst_hbm.at[pl.ds(wid * BPW, ICH)], didx.at[0])
        plsc.subcore_barrier()

        lanes = lax.iota(jnp.int32, 16)
        wk_vecs = [plsc.load_gather(wbuf, [jnp.full((16,), k, jnp.int32)])
                   for k in range(5)]

        def idx_row(c, b):
            return c.at[(b // ICH) % 2, b % ICH]

        def fire_gathers(b, p):
            srow = idx_row(sidx, b)
            drow = idx_row(didx, b)
            pltpu.async_copy(state_hbm.at[srow], xs.at[p], gsems[p])
            pltpu.async_copy(state_hbm.at[drow], xd.at[p], gsems[p])
            pltpu.async_copy(y_hbm.at[srow], ys.at[p], gsems[p])
            pltpu.async_copy(y_hbm.at[drow], yd.at[p], gsems[p])

        def wait_gathers(p):
            pltpu.make_async_copy(state_hbm.at[sidx.at[0, 0]], xs.at[p],
                                  gsems[p]).wait()
            pltpu.make_async_copy(state_hbm.at[didx.at[0, 0]], xd.at[p],
                                  gsems[p]).wait()
            pltpu.make_async_copy(y_hbm.at[sidx.at[0, 0]], ys.at[p],
                                  gsems[p]).wait()
            pltpu.make_async_copy(y_hbm.at[didx.at[0, 0]], yd.at[p],
                                  gsems[p]).wait()

        def fire_scatters(b, p):
            pltpu.async_copy(xs.at[p], acc.at[idx_row(didx, b)],
                             ssems[p], add=True)
            pltpu.async_copy(xd.at[p], acc.at[idx_row(sidx, b)],
                             ssems[p], add=True)

        def wait_scatters(p):
            pltpu.make_async_copy(xs.at[p], acc.at[didx.at[0, 0]],
                                  ssems[p]).wait()
            pltpu.make_async_copy(xd.at[p], acc.at[sidx.at[0, 0]],
                                  ssems[p]).wait()

        def compute(p):
            for g in range(B // 16):
                rows = g * 16 + lanes
                e_acc = jnp.zeros((16,), jnp.float32)
                for k in range(5):
                    col = jnp.full((16,), k, jnp.int32)
                    a = plsc.load_gather(ys.at[p], [rows, col])
                    b_ = plsc.load_gather(yd.at[p], [rows, col])
                    e_acc = e_acc + wk_vecs[k] * jnp.abs(a - b_)
                en[p, pl.ds(g * 16, 16)] = e_acc

            @plsc.parallel_loop(0, B, 1, unroll=4)
            def mbody(r):
                s = plsc.load_gather(en.at[p], [jnp.full((16,), r, jnp.int32)])
                for j in range(D // 16):
                    a = xs[p, r, pl.ds(16 * j, 16)]
                    b_ = xd[p, r, pl.ds(16 * j, 16)]
                    m = (a - b_) * s
                    xs[p, r, pl.ds(16 * j, 16)] = m
                    xd[p, r, pl.ds(16 * j, 16)] = -m

        fire_gathers(0, 0)

        def pair_body(i, carry):
            for p in range(2):
                b = 2 * i + p
                pp = 1 - p
                nb = b + 1

                @pl.when(b > 0)
                def _():
                    wait_scatters(pp)

                @pl.when(jnp.logical_and(nb < BPW, nb % ICH == 0))
                def _():
                    q = (nb // ICH) % 2
                    pltpu.sync_copy(src_hbm.at[pl.ds(wid * BPW + nb, ICH)],
                                    sidx.at[q])
                    pltpu.sync_copy(dst_hbm.at[pl.ds(wid * BPW + nb, ICH)],
                                    didx.at[q])

                @pl.when(nb < BPW)
                def _():
                    fire_gathers(nb, pp)

                wait_gathers(p)
                compute(p)
                fire_scatters(b, p)
            return carry

        lax.fori_loop(0, BPW // 2, pair_body, 0)
        wait_scatters(1)
        plsc.subcore_barrier()

        for k in range(_NCHUNK_CEIL):
            chunk = sid + k * NS

            @pl.when(chunk < NROWCHUNKS)
            def _():
                pltpu.sync_copy(acc.at[pl.ds(chunk * ZCH, ZCH)],
                                out_hbm.at[pl.ds(cid * N + chunk * ZCH, ZCH)])

    return pl.kernel(
        body,
        out_type=jax.ShapeDtypeStruct((2 * N, D), jnp.float32),
        mesh=mesh,
        scratch_types=[
            pltpu.VMEM_SHARED((N, D), jnp.float32),
            pltpu.VMEM((2, B, D), jnp.float32),
            pltpu.VMEM((2, B, D), jnp.float32),
            pltpu.VMEM((2, B, YC), jnp.float32),
            pltpu.VMEM((2, B, YC), jnp.float32),
            pltpu.VMEM((2, B), jnp.float32),
            pltpu.VMEM((2, ICH, B), jnp.int32),
            pltpu.VMEM((2, ICH, B), jnp.int32),
            pltpu.VMEM((16,), jnp.float32),
            pltpu.SemaphoreType.DMA,
            pltpu.SemaphoreType.DMA,
            pltpu.SemaphoreType.DMA,
            pltpu.SemaphoreType.DMA,
        ],
        compiler_params=pltpu.CompilerParams(needs_layout_passes=False,
                                             use_tc_tiling_on_sc=False),
    )


RB = 1000


def _proj_body(s_ref, w_ref, y_ref):
    y_ref[...] = jnp.dot(s_ref[...], w_ref[...],
                         preferred_element_type=jnp.float32)


def _update_body(s_ref, a0_ref, a1_ref, w_ref, o_ref, y_ref):
    ns = s_ref[...] - STEP * (a0_ref[...] + a1_ref[...])
    o_ref[...] = ns
    y_ref[...] = jnp.dot(ns, w_ref[...], preferred_element_type=jnp.float32)


def _make_tc_kernels():
    grid = (N // RB,)
    s_spec = pl.BlockSpec((RB, D), lambda i: (i, 0))
    w_spec = pl.BlockSpec((D, YC), lambda i: (0, 0))
    y_spec = pl.BlockSpec((RB, YC), lambda i: (i, 0))
    proj = pl.pallas_call(
        _proj_body,
        grid=grid,
        in_specs=[s_spec, w_spec],
        out_specs=y_spec,
        out_shape=jax.ShapeDtypeStruct((N, YC), jnp.float32),
    )
    a0_spec = pl.BlockSpec((RB, D), lambda i: (i, 0))
    a1_spec = pl.BlockSpec((RB, D), lambda i: (i + N // RB, 0))
    update = pl.pallas_call(
        _update_body,
        grid=grid,
        in_specs=[s_spec, a0_spec, a1_spec, w_spec],
        out_specs=[s_spec, y_spec],
        out_shape=[jax.ShapeDtypeStruct((N, D), jnp.float32),
                   jax.ShapeDtypeStruct((N, YC), jnp.float32)],
    )
    return proj, update


def kernel(x, W, bobot, edge_index):
    w = jax.nn.softmax(bobot)
    w16 = jnp.zeros((16,), jnp.float32).at[:5].set(w)
    Wp = jnp.zeros((D, YC), jnp.float32).at[:, :5].set(W)
    pad = jnp.zeros((EPAD - E,), jnp.int32)
    src = jnp.concatenate([edge_index[0], pad]).reshape(EROWS, B)
    dst = jnp.concatenate([edge_index[1], pad]).reshape(EROWS, B)

    sc_step = _sc_step_kernel()
    proj, update = _make_tc_kernels()

    state = x
    Y = proj(state, Wp)
    for _ in range(MAX_ITER):
        acc = sc_step(state, Y, src, dst, w16)
        state, Y = update(state, acc, acc, Wp)
    return state


# speedup vs baseline: 1.6395x; 1.6395x over previous
import functools

import jax
import jax.numpy as jnp
from jax import lax
from jax.experimental import pallas as pl
from jax.experimental.pallas import tpu as pltpu
from jax.experimental.pallas import tpu_sc as plsc

N = 10000
D = 128
E = 320000
MAX_ITER = 10
STEP = 0.1 / 1.5
YC = 16
NC = 2
NS = 16
NW = NC * NS
EPW = E // NW
B = 80
NBLK = EPW // B
NROWCHUNKS = N // B
_NCHUNK_CEIL = -(-NROWCHUNKS // NS)


def _sc_step_kernel():
    mesh = plsc.VectorSubcoreMesh(core_axis_name="c", subcore_axis_name="s",
                                  num_cores=NC, num_subcores=NS)

    def body(state_hbm, y_hbm, src_hbm, dst_hbm, w_hbm, out_hbm,
             acc, xs, xd, ys, yd, en, sidx, didx, wbuf, gsem, ssem):
        cid = lax.axis_index("c")
        sid = lax.axis_index("s")
        wid = sid * NC + cid

        zero16 = jnp.zeros((16,), jnp.float32)

        def zbody(r, carry):
            for j in range(D // 16):
                xs[r, pl.ds(16 * j, 16)] = zero16
            return carry

        lax.fori_loop(0, B, zbody, 0)

        for k in range(_NCHUNK_CEIL):
            chunk = sid + k * NS

            @pl.when(chunk < NROWCHUNKS)
            def _():
                pltpu.sync_copy(xs, acc.at[pl.ds(chunk * B, B)])

        pltpu.sync_copy(w_hbm, wbuf)
        plsc.subcore_barrier()

        lanes = lax.iota(jnp.int32, 16)
        wk_vecs = [plsc.load_gather(wbuf, [jnp.full((16,), k, jnp.int32)])
                   for k in range(5)]

        def blk_body(blk, carry):
            base = wid * EPW + blk * B

            @pl.when(blk > 0)
            def _():
                pltpu.make_async_copy(xs, acc.at[didx], ssem).wait()
                pltpu.make_async_copy(xd, acc.at[sidx], ssem).wait()

            pltpu.sync_copy(src_hbm.at[pl.ds(base, B)], sidx)
            pltpu.sync_copy(dst_hbm.at[pl.ds(base, B)], didx)
            cps = [pltpu.async_copy(state_hbm.at[sidx], xs, gsem),
                   pltpu.async_copy(state_hbm.at[didx], xd, gsem),
                   pltpu.async_copy(y_hbm.at[sidx], ys, gsem),
                   pltpu.async_copy(y_hbm.at[didx], yd, gsem)]
            for cp in cps:
                cp.wait()

            for g in range(B // 16):
                rows = g * 16 + lanes
                e_acc = jnp.zeros((16,), jnp.float32)
                for k in range(5):
                    col = jnp.full((16,), k, jnp.int32)
                    a = plsc.load_gather(ys, [rows, col])
                    b = plsc.load_gather(yd, [rows, col])
                    e_acc = e_acc + wk_vecs[k] * jnp.abs(a - b)
                en[pl.ds(g * 16, 16)] = e_acc

            @plsc.parallel_loop(0, B, 1, unroll=8)
            def mbody(r):
                s = plsc.load_gather(en, [jnp.full((16,), r, jnp.int32)])
                for j in range(D // 16):
                    a = xs[r, pl.ds(16 * j, 16)]
                    b = xd[r, pl.ds(16 * j, 16)]
                    m = (a - b) * s
                    xs[r, pl.ds(16 * j, 16)] = m
                    xd[r, pl.ds(16 * j, 16)] = -m

            pltpu.async_copy(xs, acc.at[didx], ssem, add=True)
            pltpu.async_copy(xd, acc.at[sidx], ssem, add=True)
            return carry

        lax.fori_loop(0, NBLK, blk_body, 0)
        pltpu.make_async_copy(xs, acc.at[didx], ssem).wait()
        pltpu.make_async_copy(xd, acc.at[sidx], ssem).wait()
        plsc.subcore_barrier()

        for k in range(_NCHUNK_CEIL):
            chunk = sid + k * NS

            @pl.when(chunk < NROWCHUNKS)
            def _():
                pltpu.sync_copy(acc.at[pl.ds(chunk * B, B)],
                                out_hbm.at[pl.ds(cid * N + chunk * B, B)])

    return pl.kernel(
        body,
        out_type=jax.ShapeDtypeStruct((2 * N, D), jnp.float32),
        mesh=mesh,
        scratch_types=[
            pltpu.VMEM_SHARED((N, D), jnp.float32),
            pltpu.VMEM((B, D), jnp.float32),
            pltpu.VMEM((B, D), jnp.float32),
            pltpu.VMEM((B, YC), jnp.float32),
            pltpu.VMEM((B, YC), jnp.float32),
            pltpu.VMEM((B,), jnp.float32),
            pltpu.VMEM((B,), jnp.int32),
            pltpu.VMEM((B,), jnp.int32),
            pltpu.VMEM((16,), jnp.float32),
            pltpu.SemaphoreType.DMA,
            pltpu.SemaphoreType.DMA,
        ],
        compiler_params=pltpu.CompilerParams(needs_layout_passes=False,
                                             use_tc_tiling_on_sc=False),
    )


RB = 1000


def _proj_body(s_ref, w_ref, y_ref):
    y_ref[...] = jnp.dot(s_ref[...], w_ref[...],
                         preferred_element_type=jnp.float32)


def _update_body(s_ref, a0_ref, a1_ref, w_ref, o_ref, y_ref):
    ns = s_ref[...] - STEP * (a0_ref[...] + a1_ref[...])
    o_ref[...] = ns
    y_ref[...] = jnp.dot(ns, w_ref[...], preferred_element_type=jnp.float32)


def _make_tc_kernels():
    grid = (N // RB,)
    s_spec = pl.BlockSpec((RB, D), lambda i: (i, 0))
    w_spec = pl.BlockSpec((D, YC), lambda i: (0, 0))
    y_spec = pl.BlockSpec((RB, YC), lambda i: (i, 0))
    proj = pl.pallas_call(
        _proj_body,
        grid=grid,
        in_specs=[s_spec, w_spec],
        out_specs=y_spec,
        out_shape=jax.ShapeDtypeStruct((N, YC), jnp.float32),
    )
    a0_spec = pl.BlockSpec((RB, D), lambda i: (i, 0))
    a1_spec = pl.BlockSpec((RB, D), lambda i: (i + N // RB, 0))
    update = pl.pallas_call(
        _update_body,
        grid=grid,
        in_specs=[s_spec, a0_spec, a1_spec, w_spec],
        out_specs=[s_spec, y_spec],
        out_shape=[jax.ShapeDtypeStruct((N, D), jnp.float32),
                   jax.ShapeDtypeStruct((N, YC), jnp.float32)],
    )
    return proj, update


def kernel(x, W, bobot, edge_index):
    w = jax.nn.softmax(bobot)
    w16 = jnp.zeros((16,), jnp.float32).at[:5].set(w)
    Wp = jnp.zeros((D, YC), jnp.float32).at[:, :5].set(W)
    src = edge_index[0]
    dst = edge_index[1]

    sc_step = _sc_step_kernel()
    proj, update = _make_tc_kernels()

    state = x
    Y = proj(state, Wp)
    for _ in range(MAX_ITER):
        acc = sc_step(state, Y, src, dst, w16)
        state, Y = update(state, acc, acc, Wp)
    return state
